# Initial kernel scaffold; baseline (speedup 1.0000x reference)
import jax
import jax.numpy as jnp
from jax.experimental import pallas as pl


def kernel(xs, hc0, w_packed, b_packed):
    raise NotImplementedError("write your pallas kernel here")


# speedup vs baseline: 2.0127x; 2.0127x over previous
import functools

import jax
import jax.numpy as jnp
from jax.experimental import pallas as pl
from jax.experimental.pallas import tpu as pltpu


def _round_up(v, m):
    return ((v + m - 1) // m) * m


def _make_body(T, H, W, Cin, hid, kh, kw, Cpad, TR):
    ph, pw = kh // 2, kw // 2
    NR = H // TR
    f32 = jnp.float32
    bf16 = jnp.bfloat16

    def body(x_ref, hc0_ref, w_ref, b_ref, out_ref, comb_ref, h_ref, c_ref):
        comb_ref[...] = jnp.zeros_like(comb_ref)
        hc0 = hc0_ref[0].astype(f32)
        h_ref[...] = hc0[..., :hid]
        c_ref[...] = hc0[..., hid:2 * hid]
        b = b_ref[0].astype(f32)

        def step(t, carry):
            comb_ref[ph:ph + H, pw:pw + W, 0:hid] = h_ref[...].astype(bf16)
            comb_ref[ph:ph + H, pw:pw + W, hid:hid + Cin] = x_ref[0, t]

            def row_tile(r, cc):
                r0 = pl.multiple_of(r * TR, TR)
                pieces = [comb_ref[pl.ds(r0 + ki, TR), kj:kj + W, :]
                          for ki in range(kh) for kj in range(kw)]
                patches = jnp.concatenate(pieces, axis=-1)

                acc = jax.lax.dot_general(
                    patches, w_ref[...],
                    dimension_numbers=(((2,), (0,)), ((), ())),
                    preferred_element_type=f32)
                acc = acc + b

                lane = jax.lax.broadcasted_iota(jnp.int32, acc.shape, 2)
                is_sig = lane < 3 * hid
                th = jnp.tanh(jnp.where(is_sig, 0.5 * acc, acc))
                act = jnp.where(is_sig, 0.5 * (th + 1.0), th)

                i = act[..., 0 * hid:1 * hid]
                f = act[..., 1 * hid:2 * hid]
                o = act[..., 2 * hid:3 * hid]
                g = act[..., 3 * hid:4 * hid]

                c_cur = c_ref[pl.ds(r0, TR), :, :]
                c_next = f * c_cur + i * g
                h_next = o * jnp.tanh(c_next)

                c_ref[pl.ds(r0, TR), :, :] = c_next
                h_ref[pl.ds(r0, TR), :, :] = h_next
                return cc

            jax.lax.fori_loop(0, NR, row_tile, 0, unroll=True)
            return carry

        jax.lax.fori_loop(0, T, step, 0)

        out_ref[0] = jnp.concatenate(
            [h_ref[...], c_ref[...]], axis=-1).astype(out_ref.dtype)

    return body


@functools.partial(jax.jit, static_argnames=("input_dim", "hidden_dim",
                                             "kernel_size"))
def _convlstm_seq(xs, hc0, w_packed, b_packed, *,
                  input_dim, hidden_dim, kernel_size):
    B, T, H, W, Cin = xs.shape
    hid = hidden_dim
    kh, kw = kernel_size
    C = Cin + hid
    Cpad = _round_up(C, 64)
    K = kh * kw * Cpad
    assert Cin == input_dim
    assert hc0.shape == (B, H, W, 2 * hid)
    assert w_packed.shape == (K, 4 * hid)

    TR = next((tr for tr in (8, 4, 2, 1) if H % tr == 0), 1)

    ph, pw = kh // 2, kw // 2
    Hp, Wp = H + 2 * ph, W + 2 * pw

    xs_bf = xs.astype(jnp.bfloat16)
    w_bf = w_packed.astype(jnp.bfloat16)

    body = _make_body(T, H, W, Cin, hid, kh, kw, Cpad, TR)
    return pl.pallas_call(
        body,
        out_shape=jax.ShapeDtypeStruct((B, H, W, 2 * hid), xs.dtype),
        grid_spec=pltpu.PrefetchScalarGridSpec(
            num_scalar_prefetch=0,
            grid=(B,),
            in_specs=[
                pl.BlockSpec((1, T, H, W, Cin), lambda b: (b, 0, 0, 0, 0)),
                pl.BlockSpec((1, H, W, 2 * hid), lambda b: (b, 0, 0, 0)),
                pl.BlockSpec((K, 4 * hid), lambda b: (0, 0)),
                pl.BlockSpec((1, 4 * hid), lambda b: (0, 0)),
            ],
            out_specs=pl.BlockSpec((1, H, W, 2 * hid), lambda b: (b, 0, 0, 0)),
            scratch_shapes=[
                pltpu.VMEM((Hp, Wp, Cpad), jnp.bfloat16),
                pltpu.VMEM((H, W, hid), jnp.float32),
                pltpu.VMEM((H, W, hid), jnp.float32),
            ]),
        compiler_params=pltpu.CompilerParams(
            dimension_semantics=("parallel",)),
    )(xs_bf, hc0, w_bf, b_packed)


def kernel(xs, hc0, w_packed, b_packed):
    return _convlstm_seq(xs, hc0, w_packed, b_packed,
                         input_dim=64, hidden_dim=64, kernel_size=(3, 3))
